# baseline (speedup 1.0000x reference)
import functools
import jax
import jax.numpy as jnp
from jax import lax
from jax.experimental import pallas as pl
from jax.experimental.pallas import tpu as pltpu, tpu_sc as plsc

B, S, D = 4, 8192, 768
L = 16
W = 128
NW = 32
RPW = (B * S * D // W) // NW
CR = 192
G = RPW // CR
K = G // 2

_mesh = plsc.VectorSubcoreMesh(core_axis_name="c", subcore_axis_name="s")


@functools.partial(
    pl.kernel,
    mesh=_mesh,
    out_type=jax.ShapeDtypeStruct((B * S * D // W, W), jnp.float32),
    scratch_types=[
        pltpu.VMEM((CR, W), jnp.float32), pltpu.VMEM((CR, W), jnp.float32),
        pltpu.VMEM((CR, W), jnp.float32), pltpu.VMEM((CR, W), jnp.float32),
        pltpu.SemaphoreType.DMA, pltpu.SemaphoreType.DMA,
        pltpu.SemaphoreType.DMA, pltpu.SemaphoreType.DMA,
    ],
)
def _sc_add(x_hbm, t_hbm, out_hbm,
            xb0, xb1, tb0, tb1,
            si0, si1, so0, so1):
    w = lax.axis_index("c") * 16 + lax.axis_index("s")
    x_base = w * RPW
    t_base = lax.rem(w, 8) * RPW
    xbs = (xb0, xb1)
    tbs = (tb0, tb1)
    sis = (si0, si1)
    sos = (so0, so1)

    def start_in(c, p):
        o = c * CR
        pltpu.async_copy(x_hbm.at[pl.ds(x_base + o, CR), :], xbs[p], sis[p])
        pltpu.async_copy(t_hbm.at[pl.ds(t_base + o, CR), :], tbs[p], sis[p])

    def wait_in(p):
        pltpu.make_async_copy(x_hbm.at[pl.ds(0, CR), :], xbs[p], sis[p]).wait()
        pltpu.make_async_copy(t_hbm.at[pl.ds(0, CR), :], tbs[p], sis[p]).wait()

    def start_out(c, p):
        pltpu.async_copy(xbs[p], out_hbm.at[pl.ds(x_base + c * CR, CR), :],
                         sos[p])

    def wait_out(p):
        pltpu.make_async_copy(xbs[p], out_hbm.at[pl.ds(0, CR), :],
                              sos[p]).wait()

    def compute(p):
        xb, tb = xbs[p], tbs[p]

        def rows2(i, _):
            r = i * 2
            for rr in (0, 1):
                for j in range(W // L):
                    sl = pl.ds(j * L, L)
                    xb[r + rr, sl] = xb[r + rr, sl] + tb[r + rr, sl]
            return 0

        lax.fori_loop(0, CR // 2, rows2, 0)

    start_in(0, 0)
    start_in(1, 1)

    def macro(k, _):
        c = k * 2
        wait_in(0)
        compute(0)
        start_out(c, 0)
        wait_in(1)
        compute(1)
        start_out(c + 1, 1)

        @pl.when(k < K - 1)
        def _():
            wait_out(0)
            start_in(c + 2, 0)
            wait_out(1)
            start_in(c + 3, 1)

        return 0

    lax.fori_loop(0, K, macro, 0)
    wait_out(0)
    wait_out(1)


def kernel(x, embed_table):
    Bx, Sx, Dx = x.shape
    xf = x.reshape(Bx * Sx * Dx // W, W)
    tf = embed_table.reshape(-1, W)
    out = _sc_add(xf, tf)
    return out.reshape(Bx, Sx, Dx)


# speedup vs baseline: 1.0866x; 1.0866x over previous
import functools
import jax
import jax.numpy as jnp
from jax import lax
from jax.experimental import pallas as pl
from jax.experimental.pallas import tpu as pltpu, tpu_sc as plsc

B, S, D = 4, 8192, 768
L = 16
W = 128
RW = D // W
NW = 32
PPW = S // NW
CP = 32
CR = CP * RW
NG = PPW // CP
NU = NG * B

_mesh = plsc.VectorSubcoreMesh(core_axis_name="c", subcore_axis_name="s")


@functools.partial(
    pl.kernel,
    mesh=_mesh,
    out_type=jax.ShapeDtypeStruct((B * S * D // W, W), jnp.float32),
    scratch_types=[
        pltpu.VMEM((CR, W), jnp.float32), pltpu.VMEM((CR, W), jnp.float32),
        pltpu.VMEM((CR, W), jnp.float32), pltpu.VMEM((CR, W), jnp.float32),
        pltpu.SemaphoreType.DMA, pltpu.SemaphoreType.DMA,
        pltpu.SemaphoreType.DMA, pltpu.SemaphoreType.DMA,
        pltpu.SemaphoreType.DMA, pltpu.SemaphoreType.DMA,
    ],
)
def _sc_add(x_hbm, t_hbm, out_hbm,
            xb0, xb1, tb0, tb1,
            si0, si1, so0, so1, st0, st1):
    w = lax.axis_index("c") * 16 + lax.axis_index("s")
    t_base = w * (PPW * RW)
    xbs = (xb0, xb1)
    tbs = (tb0, tb1)
    sis = (si0, si1)
    sos = (so0, so1)
    sts = (st0, st1)

    def u_off(u):
        gc = lax.div(u, B)
        b = lax.rem(u, B)
        return b * (S * RW) + t_base + gc * CR

    def start_x(u, p):
        pltpu.async_copy(x_hbm.at[pl.ds(u_off(u), CR), :], xbs[p], sis[p])

    def wait_x(p):
        pltpu.make_async_copy(x_hbm.at[pl.ds(0, CR), :], xbs[p], sis[p]).wait()

    def start_t(gc, q):
        pltpu.async_copy(t_hbm.at[pl.ds(t_base + gc * CR, CR), :],
                         tbs[q], sts[q])

    def wait_t(q):
        pltpu.make_async_copy(t_hbm.at[pl.ds(0, CR), :], tbs[q],
                              sts[q]).wait()

    def start_out(u, p):
        pltpu.async_copy(xbs[p], out_hbm.at[pl.ds(u_off(u), CR), :], sos[p])

    def wait_out(p):
        pltpu.make_async_copy(xbs[p], out_hbm.at[pl.ds(0, CR), :],
                              sos[p]).wait()

    def compute(p, q):
        xb, tb = xbs[p], tbs[q]

        def rows2(i, _):
            r = i * 2
            for rr in (0, 1):
                for j in range(W // L):
                    sl = pl.ds(j * L, L)
                    xb[r + rr, sl] = xb[r + rr, sl] + tb[r + rr, sl]
            return 0

        lax.fori_loop(0, CR // 2, rows2, 0)

    start_t(0, 0)
    start_x(0, 0)
    start_x(1, 1)

    def do_chunk(gc, q):
        wait_t(q)

        @pl.when(gc + 1 < NG)
        def _():
            start_t(gc + 1, 1 - q)

        for jp in range(B // 2):
            uA = gc * B + 2 * jp
            wait_x(0)
            compute(0, q)
            start_out(uA, 0)
            wait_x(1)
            compute(1, q)
            start_out(uA + 1, 1)

            @pl.when(uA + 2 < NU)
            def _():
                wait_out(0)
                start_x(uA + 2, 0)

            @pl.when(uA + 3 < NU)
            def _():
                wait_out(1)
                start_x(uA + 3, 1)

    def macro(m, _):
        do_chunk(2 * m, 0)
        do_chunk(2 * m + 1, 1)
        return 0

    lax.fori_loop(0, NG // 2, macro, 0)
    wait_out(0)
    wait_out(1)


def kernel(x, embed_table):
    Bx, Sx, Dx = x.shape
    xf = x.reshape(Bx * Sx * Dx // W, W)
    tf = embed_table.reshape(-1, W)
    out = _sc_add(xf, tf)
    return out.reshape(Bx, Sx, Dx)


# speedup vs baseline: 5.2829x; 4.8620x over previous
import jax
import jax.numpy as jnp
from jax.experimental import pallas as pl

_BS = 1024


def _add_kernel(x_ref, t_ref, o_ref):
    o_ref[...] = x_ref[...] + t_ref[...]


def kernel(x, embed_table):
    B, S, D = x.shape
    return pl.pallas_call(
        _add_kernel,
        grid=(S // _BS,),
        in_specs=[
            pl.BlockSpec((B, _BS, D), lambda s: (0, s, 0)),
            pl.BlockSpec((_BS, D), lambda s: (s, 0)),
        ],
        out_specs=pl.BlockSpec((B, _BS, D), lambda s: (0, s, 0)),
        out_shape=jax.ShapeDtypeStruct((B, S, D), x.dtype),
    )(x, embed_table)
